# baseline (speedup 1.0000x reference)
import functools

import jax
import jax.numpy as jnp
import numpy as np
from jax.experimental import pallas as pl
from jax.experimental.pallas import tpu as pltpu

VMEM_LIMIT = 64 * 1024 * 1024


def _gcn_kernel(i_ref, j_ref, e_ref, x_ref, blk_ref, w1_ref, w2_ref,
                b1_ref, b2_ref, out_ref,
                tri_res, s1_buf, s1t_buf, s2_buf, s2t_buf,
                h1d, h1t, outd, outt, *, nb, bs, nh1, nh2, n_steps):
    t = pl.program_id(0)

    @pl.when(t == 0)
    def _():
        s1 = jnp.dot(x_ref[...].astype(jnp.bfloat16), w1_ref[...],
                     preferred_element_type=jnp.float32).astype(jnp.bfloat16)
        for m in range(nb):
            s1_buf[m] = s1[m * bs:(m + 1) * bs, :]
            s1t_buf[m] = s1[m * bs:(m + 1) * bs, :].T
        h1d[...] = jnp.zeros_like(h1d)
        h1t[...] = jnp.zeros_like(h1t)
        outd[...] = jnp.zeros_like(outd)
        outt[...] = jnp.zeros_like(outt)

    @pl.when(t > 0)
    def _():
        i = i_ref[t]
        j = j_ref[t]
        a = blk_ref[...].astype(jnp.bfloat16)
        tri_res[t - 1] = a

        h1d[i] += jnp.dot(a, s1_buf[j], preferred_element_type=jnp.float32)
        @pl.when(i != j)
        def _():
            h1t[j] += jnp.dot(s1t_buf[i], a,
                              preferred_element_type=jnp.float32)

        @pl.when(e_ref[t] == 1)
        def _():
            r = i
            h1_r = h1d[r] + h1t[r].T + b1_ref[...]
            h1_r = jnp.maximum(h1_r, 0.0).astype(jnp.bfloat16)
            s2_r = jnp.dot(h1_r, w2_ref[...],
                           preferred_element_type=jnp.float32
                           ).astype(jnp.bfloat16)
            s2_buf[r] = s2_r
            s2t_buf[r] = s2_r.T
            for i2 in range(nb):
                base = i2 * nb - (i2 * (i2 - 1)) // 2 - i2
                @pl.when(i2 <= r)
                def _():
                    blk = tri_res[base + r]
                    outd[i2] += jnp.dot(blk, s2_buf[r],
                                        preferred_element_type=jnp.float32)
                    @pl.when(i2 != r)
                    def _():
                        outt[r] += jnp.dot(s2t_buf[i2], blk,
                                           preferred_element_type=jnp.float32)

        @pl.when(t == n_steps - 1)
        def _():
            b2 = b2_ref[...]
            for m in range(nb):
                out_ref[m * bs:(m + 1) * bs, :] = outd[m] + outt[m].T + b2


def kernel(feature, adj, w1, b1, w2, b2):
    n, nfeat = feature.shape
    nhid1 = w1.shape[1]
    nhid2 = w2.shape[1]

    w1_bf = w1.astype(jnp.bfloat16)
    w2_bf = w2.astype(jnp.bfloat16)
    b1_2d = b1.reshape(1, nhid1).astype(jnp.float32)
    b2_2d = b2.reshape(1, nhid2).astype(jnp.float32)

    bs = 512
    nb = n // bs
    tri = [(i, j) for i in range(nb) for j in range(i, nb)]
    n_blocks = len(tri)
    n_steps = n_blocks + 1

    ii = np.array([0] + [ij[0] for ij in tri], dtype=np.int32)
    jj = np.array([0] + [ij[1] for ij in tri], dtype=np.int32)
    ee = np.array([0] + [1 if ij[1] == nb - 1 else 0 for ij in tri],
                  dtype=np.int32)

    body = functools.partial(_gcn_kernel, nb=nb, bs=bs, nh1=nhid1,
                             nh2=nhid2, n_steps=n_steps)
    out = pl.pallas_call(
        body,
        out_shape=jax.ShapeDtypeStruct((n, nhid2), jnp.float32),
        grid_spec=pltpu.PrefetchScalarGridSpec(
            num_scalar_prefetch=3,
            grid=(n_steps,),
            in_specs=[
                pl.BlockSpec((n, nfeat), lambda t, I, J, E: (0, 0)),
                pl.BlockSpec((bs, bs), lambda t, I, J, E: (I[t], J[t])),
                pl.BlockSpec((nfeat, nhid1), lambda t, I, J, E: (0, 0)),
                pl.BlockSpec((nhid1, nhid2), lambda t, I, J, E: (0, 0)),
                pl.BlockSpec((1, nhid1), lambda t, I, J, E: (0, 0)),
                pl.BlockSpec((1, nhid2), lambda t, I, J, E: (0, 0)),
            ],
            out_specs=pl.BlockSpec((n, nhid2), lambda t, I, J, E: (0, 0)),
            scratch_shapes=[
                pltpu.VMEM((n_blocks, bs, bs), jnp.bfloat16),
                pltpu.VMEM((nb, bs, nhid1), jnp.bfloat16),
                pltpu.VMEM((nb, nhid1, bs), jnp.bfloat16),
                pltpu.VMEM((nb, bs, nhid2), jnp.bfloat16),
                pltpu.VMEM((nb, nhid2, bs), jnp.bfloat16),
                pltpu.VMEM((nb, bs, nhid1), jnp.float32),
                pltpu.VMEM((nb, nhid1, bs), jnp.float32),
                pltpu.VMEM((nb, bs, nhid2), jnp.float32),
                pltpu.VMEM((nb, nhid2, bs), jnp.float32),
            ],
        ),
        compiler_params=pltpu.CompilerParams(
            dimension_semantics=("arbitrary",),
            vmem_limit_bytes=VMEM_LIMIT),
    )(jnp.asarray(ii), jnp.asarray(jj), jnp.asarray(ee),
      feature, adj, w1_bf, w2_bf, b1_2d, b2_2d)
    return out


# speedup vs baseline: 1.5347x; 1.5347x over previous
import functools

import jax
import jax.numpy as jnp
import numpy as np
from jax.experimental import pallas as pl
from jax.experimental.pallas import tpu as pltpu

VMEM_LIMIT = 64 * 1024 * 1024


def _gcn_kernel(i_ref, j_ref, e_ref, x_ref, blk_ref, w1_ref, w2_ref,
                b1_ref, b2_ref, out_ref,
                tri_res, s1_buf, s1t_buf, s2_buf, s2t_buf,
                h1d, h1t, outd, outt, *, nb, bs, nh1, nh2, n_steps):
    t = pl.program_id(0)

    @pl.when(t == 0)
    def _():
        s1 = jnp.dot(x_ref[...].astype(jnp.bfloat16), w1_ref[...],
                     preferred_element_type=jnp.float32).astype(jnp.bfloat16)
        for m in range(nb):
            s1_buf[m] = s1[m * bs:(m + 1) * bs, :]
            s1t_buf[m] = s1[m * bs:(m + 1) * bs, :].T
        h1d[...] = jnp.zeros_like(h1d)
        h1t[...] = jnp.zeros_like(h1t)
        outd[...] = jnp.zeros_like(outd)
        outt[...] = jnp.zeros_like(outt)

    @pl.when(t > 0)
    def _():
        i = i_ref[t]
        j = j_ref[t]
        a = blk_ref[...].astype(jnp.bfloat16)
        tri_res[t - 1] = a

        h1d[i] += jnp.dot(a, s1_buf[j], preferred_element_type=jnp.float32)
        @pl.when(i != j)
        def _():
            h1t[j] += jnp.dot(s1t_buf[i], a,
                              preferred_element_type=jnp.float32)

        @pl.when(e_ref[t] == 1)
        def _():
            r = i
            h1_r = h1d[r] + h1t[r].T + b1_ref[...]
            h1_r = jnp.maximum(h1_r, 0.0).astype(jnp.bfloat16)
            s2_r = jnp.dot(h1_r, w2_ref[...],
                           preferred_element_type=jnp.float32
                           ).astype(jnp.bfloat16)
            s2_buf[r] = s2_r
            s2t_buf[r] = s2_r.T
            for i2 in range(nb):
                base = i2 * nb - (i2 * (i2 - 1)) // 2 - i2
                @pl.when(i2 <= r)
                def _():
                    blk = tri_res[base + r]
                    outd[i2] += jnp.dot(blk, s2_buf[r],
                                        preferred_element_type=jnp.float32)
                    @pl.when(i2 != r)
                    def _():
                        outt[r] += jnp.dot(s2t_buf[i2], blk,
                                           preferred_element_type=jnp.float32)

        @pl.when(t == n_steps - 1)
        def _():
            b2 = b2_ref[...]
            for m in range(nb):
                out_ref[m * bs:(m + 1) * bs, :] = outd[m] + outt[m].T + b2


def kernel(feature, adj, w1, b1, w2, b2):
    n, nfeat = feature.shape
    nhid1 = w1.shape[1]
    nhid2 = w2.shape[1]

    w1_bf = w1.astype(jnp.bfloat16)
    w2_bf = w2.astype(jnp.bfloat16)
    b1_2d = b1.reshape(1, nhid1).astype(jnp.float32)
    b2_2d = b2.reshape(1, nhid2).astype(jnp.float32)

    bs = 1024
    nb = n // bs
    tri = [(i, j) for i in range(nb) for j in range(i, nb)]
    n_blocks = len(tri)
    n_steps = n_blocks + 1

    ii = np.array([0] + [ij[0] for ij in tri], dtype=np.int32)
    jj = np.array([0] + [ij[1] for ij in tri], dtype=np.int32)
    ee = np.array([0] + [1 if ij[1] == nb - 1 else 0 for ij in tri],
                  dtype=np.int32)

    body = functools.partial(_gcn_kernel, nb=nb, bs=bs, nh1=nhid1,
                             nh2=nhid2, n_steps=n_steps)
    out = pl.pallas_call(
        body,
        out_shape=jax.ShapeDtypeStruct((n, nhid2), jnp.float32),
        grid_spec=pltpu.PrefetchScalarGridSpec(
            num_scalar_prefetch=3,
            grid=(n_steps,),
            in_specs=[
                pl.BlockSpec((n, nfeat), lambda t, I, J, E: (0, 0)),
                pl.BlockSpec((bs, bs), lambda t, I, J, E: (I[t], J[t])),
                pl.BlockSpec((nfeat, nhid1), lambda t, I, J, E: (0, 0)),
                pl.BlockSpec((nhid1, nhid2), lambda t, I, J, E: (0, 0)),
                pl.BlockSpec((1, nhid1), lambda t, I, J, E: (0, 0)),
                pl.BlockSpec((1, nhid2), lambda t, I, J, E: (0, 0)),
            ],
            out_specs=pl.BlockSpec((n, nhid2), lambda t, I, J, E: (0, 0)),
            scratch_shapes=[
                pltpu.VMEM((n_blocks, bs, bs), jnp.bfloat16),
                pltpu.VMEM((nb, bs, nhid1), jnp.bfloat16),
                pltpu.VMEM((nb, nhid1, bs), jnp.bfloat16),
                pltpu.VMEM((nb, bs, nhid2), jnp.bfloat16),
                pltpu.VMEM((nb, nhid2, bs), jnp.bfloat16),
                pltpu.VMEM((nb, bs, nhid1), jnp.float32),
                pltpu.VMEM((nb, nhid1, bs), jnp.float32),
                pltpu.VMEM((nb, bs, nhid2), jnp.float32),
                pltpu.VMEM((nb, nhid2, bs), jnp.float32),
            ],
        ),
        compiler_params=pltpu.CompilerParams(
            dimension_semantics=("arbitrary",),
            vmem_limit_bytes=VMEM_LIMIT),
    )(jnp.asarray(ii), jnp.asarray(jj), jnp.asarray(ee),
      feature, adj, w1_bf, w2_bf, b1_2d, b2_2d)
    return out
